# baseline (speedup 1.0000x reference)
import functools

import jax
import jax.numpy as jnp
from jax import lax
from jax.experimental import pallas as pl
from jax.experimental.pallas import tpu as pltpu
from jax.experimental.pallas import tpu_sc as plsc

_HID = 64
_SCALE = 8.0

_NC = 2
_NS = 16
_NW = _NC * _NS
_L = 16

_CHUNK = 128


def _make_kernel(nrows, ncols):
    npair = ncols // 2
    assert npair * 2 == ncols
    mesh = plsc.VectorSubcoreMesh(
        core_axis_name="c", subcore_axis_name="s",
        num_cores=_NC, num_subcores=_NS,
    )

    @functools.partial(
        pl.kernel,
        out_type=jax.ShapeDtypeStruct((nrows, ncols * _HID), jnp.float32),
        mesh=mesh,
        scratch_types=(
            [
                pltpu.VMEM((ncols, _CHUNK), jnp.int32),
                pltpu.VMEM((ncols, _CHUNK), jnp.int32),
            ]
            + [pltpu.VMEM((_CHUNK, 128), jnp.float32) for _ in range(4)]
            + [pltpu.VMEM((_CHUNK, 128), jnp.float32) for _ in range(2)]
            + [pltpu.SemaphoreType.DMA for _ in range(4)]
            + [pltpu.SemaphoreType.DMA for _ in range(2)]
        ),
        compiler_params=pltpu.CompilerParams(needs_layout_passes=False),
    )
    def emb_kernel(pairs_hbm, idxt_hbm, out_hbm, idx_v, par_v, *scratch):
        gbuf = scratch[0:4]
        obuf = scratch[4:6]
        gsem = scratch[6:10]
        ssem = scratch[10:12]
        wid = lax.axis_index("s") * _NC + lax.axis_index("c")
        i0 = wid * _CHUNK
        pltpu.sync_copy(idxt_hbm.at[:, pl.ds(i0, _CHUNK)], idx_v)

        @pl.loop(0, ncols)
        def _split(j):
            for k in range(_CHUNK // _L):
                sl = pl.ds(k * _L, _L)
                v = idx_v[j, sl]
                idx_v[j, sl] = v >> 1
                par_v[j, sl] = (v & 1) * _HID

        for j in range(4):
            pltpu.async_copy(pairs_hbm.at[idx_v.at[j]], gbuf[j], gsem[j])

        def _assemble(u, h, src, dst):
            j = 2 * u + h

            @pl.loop(0, _CHUNK // _L)
            def _rowgrp(g):
                r0 = g * _L
                parv = par_v[j, pl.ds(r0, _L)]
                for k in range(_L):
                    off = parv[k]
                    for m in range(_HID // _L):
                        vals = src[r0 + k, pl.ds(off + m * _L, _L)]
                        dst[r0 + k, pl.ds(h * _HID + m * _L, _L)] = (
                            vals * _SCALE)

        def _step(u, e, prefetch, drain_prev):
            b0, b1, ob = 2 * e, 2 * e + 1, e
            pltpu.make_async_copy(
                pairs_hbm.at[idx_v.at[2 * u]], gbuf[b0], gsem[b0]).wait()
            pltpu.make_async_copy(
                pairs_hbm.at[idx_v.at[2 * u + 1]], gbuf[b1], gsem[b1]).wait()

            if drain_prev:
                pltpu.make_async_copy(
                    obuf[ob],
                    out_hbm.at[pl.ds(i0, _CHUNK), pl.ds((u - 2) * 128, 128)],
                    ssem[ob]).wait()

            _assemble(u, 0, gbuf[b0], obuf[ob])
            _assemble(u, 1, gbuf[b1], obuf[ob])

            pltpu.async_copy(
                obuf[ob],
                out_hbm.at[pl.ds(i0, _CHUNK), pl.ds(u * 128, 128)],
                ssem[ob])

            if prefetch:
                def _fire(u=u, b0=b0, b1=b1):
                    pltpu.async_copy(
                        pairs_hbm.at[idx_v.at[2 * u + 4]], gbuf[b0], gsem[b0])
                    pltpu.async_copy(
                        pairs_hbm.at[idx_v.at[2 * u + 5]], gbuf[b1], gsem[b1])

                if isinstance(u, int):
                    _fire()
                else:
                    pl.when(u < npair - 2)(_fire)

        _step(0, 0, True, False)
        _step(1, 1, True, False)

        @pl.loop(1, npair // 2)
        def _round(t):
            for e in range(2):
                _step(2 * t + e, e, True, True)

        _step(npair - 1, (npair - 1) % 2, False, True)

        for u in (npair - 2, npair - 1):
            pltpu.make_async_copy(
                obuf[u % 2],
                out_hbm.at[pl.ds(i0, _CHUNK), pl.ds(u * 128, 128)],
                ssem[u % 2]).wait()

    return emb_kernel


def _make_relayout(vocab):
    nblk = vocab // 128
    tail = vocab - nblk * 128
    per_w = nblk // _NW
    extra = nblk - per_w * _NW
    mesh = plsc.VectorSubcoreMesh(
        core_axis_name="c", subcore_axis_name="s",
        num_cores=_NC, num_subcores=_NS,
    )

    @functools.partial(
        pl.kernel,
        out_type=jax.ShapeDtypeStruct((vocab // 2, 2 * _HID), jnp.float32),
        mesh=mesh,
        scratch_types=(
            [pltpu.VMEM((_HID, 128), jnp.float32) for _ in range(2)]
            + [pltpu.VMEM((_HID, 128), jnp.float32) for _ in range(2)]
            + [pltpu.SemaphoreType.DMA for _ in range(4)]
        ),
        compiler_params=pltpu.CompilerParams(needs_layout_passes=False),
    )
    def relayout_kernel(tt_hbm, tailp_hbm, pairs_hbm, ib0, ib1, ob0, ob1,
                        *sems):
        ibuf = (ib0, ib1)
        obuf = (ob0, ob1)
        gsem = sems[0:2]
        ssem = sems[2:4]
        wid = lax.axis_index("s") * _NC + lax.axis_index("c")
        start = wid * per_w + jnp.minimum(wid, extra)
        rows = [lax.iota(jnp.int32, _L) + 16 * mm for mm in range(4)]

        def _load(b, e):
            pltpu.async_copy(
                tt_hbm.at[:, pl.ds(b * 128, 128)], ibuf[e], gsem[e])

        def _wait_load(b, e):
            pltpu.make_async_copy(
                tt_hbm.at[:, pl.ds(b * 128, 128)], ibuf[e], gsem[e]).wait()

        def _store(b, e):
            pltpu.async_copy(
                obuf[e], pairs_hbm.at[pl.ds(b * 64, 64), :], ssem[e])

        def _wait_store(b, e):
            pltpu.make_async_copy(
                obuf[e], pairs_hbm.at[pl.ds(b * 64, 64), :], ssem[e]).wait()

        def _transpose(src, dst, nq):
            @pl.loop(0, nq // 4)
            def _qgrp(qg):
                for qq in range(4):
                    q = qg * 4 + qq
                    c0 = jnp.full((_L,), 2 * q, jnp.int32)
                    for m in range(8):
                        vals = plsc.load_gather(
                            src, [rows[m % 4], c0 + (m // 4)])
                        dst[q, pl.ds(16 * m, _L)] = vals

        nround = per_w // 2
        _load(start, 0)
        _load(start + 1, 1)

        @pl.loop(0, nround)
        def _round(t):
            for e in range(2):
                b = start + 2 * t + e
                _wait_load(b, e)

                @pl.when(t >= 1)
                def _(b=b, e=e):
                    _wait_store(b - 2, e)

                _transpose(ibuf[e], obuf[e], 64)
                _store(b, e)

                @pl.when(t < nround - 1)
                def _(b=b, e=e):
                    _load(b + 2, e)

        for e in range(2):
            _wait_store(start + per_w - 2 + e, e)

        @pl.when(wid < extra)
        def _():
            b = start + per_w
            pltpu.sync_copy(tt_hbm.at[:, pl.ds(b * 128, 128)], ibuf[0])
            _transpose(ibuf[0], obuf[0], 64)
            pltpu.sync_copy(obuf[0], pairs_hbm.at[pl.ds(b * 64, 64), :])

        if tail:
            @pl.when(wid == _NW - 1)
            def _():
                pltpu.sync_copy(tailp_hbm, obuf[1].at[pl.ds(0, tail // 2), :])
                pltpu.sync_copy(
                    obuf[1].at[pl.ds(0, tail // 2), :],
                    pairs_hbm.at[pl.ds(nblk * 64, tail // 2), :])

    return relayout_kernel


def kernel(x, emb_table):
    nrows, ncols = x.shape
    vocab, hid = emb_table.shape
    assert nrows == _NW * _CHUNK and hid == _HID
    xt = jnp.swapaxes(x.astype(jnp.int32), 0, 1)
    tt = jnp.swapaxes(emb_table, 0, 1)
    nblk = vocab // 128
    tailp = emb_table[nblk * 128:].reshape(-1, 2 * _HID)
    pairs = _make_relayout(vocab)(tt, tailp)
    out = _make_kernel(nrows, ncols)(pairs, xt)
    return out.reshape(nrows, ncols, _HID)


# speedup vs baseline: 1.4655x; 1.4655x over previous
import functools

import jax
import jax.numpy as jnp
from jax import lax
from jax.experimental import pallas as pl
from jax.experimental.pallas import tpu as pltpu
from jax.experimental.pallas import tpu_sc as plsc

_HID = 64
_SCALE = 8.0

_NC = 2
_NS = 16
_NW = _NC * _NS
_L = 16

_CHUNK = 128


def _make_kernel(nrows, ncols):
    npair = ncols // 2
    assert npair * 2 == ncols
    mesh = plsc.VectorSubcoreMesh(
        core_axis_name="c", subcore_axis_name="s",
        num_cores=_NC, num_subcores=_NS,
    )

    @functools.partial(
        pl.kernel,
        out_type=jax.ShapeDtypeStruct((nrows, ncols * _HID), jnp.float32),
        mesh=mesh,
        scratch_types=(
            [
                pltpu.VMEM((ncols, _CHUNK), jnp.int32),
                pltpu.VMEM((ncols, _CHUNK), jnp.int32),
            ]
            + [pltpu.VMEM((_CHUNK, 128), jnp.float32) for _ in range(4)]
            + [pltpu.VMEM((_CHUNK, 128), jnp.float32) for _ in range(2)]
            + [pltpu.SemaphoreType.DMA for _ in range(4)]
            + [pltpu.SemaphoreType.DMA for _ in range(2)]
        ),
        compiler_params=pltpu.CompilerParams(needs_layout_passes=False),
    )
    def emb_kernel(pairs_hbm, idxt_hbm, out_hbm, idx_v, par_v, *scratch):
        gbuf = scratch[0:4]
        obuf = scratch[4:6]
        gsem = scratch[6:10]
        ssem = scratch[10:12]
        wid = lax.axis_index("s") * _NC + lax.axis_index("c")
        i0 = wid * _CHUNK
        pltpu.sync_copy(idxt_hbm.at[:, pl.ds(i0, _CHUNK)], idx_v)

        @pl.loop(0, ncols)
        def _split(j):
            for k in range(_CHUNK // _L):
                sl = pl.ds(k * _L, _L)
                v = idx_v[j, sl]
                idx_v[j, sl] = v >> 1
                par_v[j, sl] = (v & 1) * _HID

        for j in range(4):
            pltpu.async_copy(pairs_hbm.at[idx_v.at[j]], gbuf[j], gsem[j])

        def _assemble(u, h, src, dst):
            j = 2 * u + h

            @pl.loop(0, _CHUNK // _L)
            def _rowgrp(g):
                r0 = g * _L
                parv = par_v[j, pl.ds(r0, _L)]
                for k2 in range(_L // 4):
                    vals = []
                    for k in range(k2 * 4, k2 * 4 + 4):
                        off = parv[k]
                        for m in range(_HID // _L):
                            vals.append(
                                src[r0 + k, pl.ds(off + m * _L, _L)]
                                * _SCALE)
                    for ki in range(4):
                        k = k2 * 4 + ki
                        for m in range(_HID // _L):
                            dst[r0 + k, pl.ds(h * _HID + m * _L, _L)] = (
                                vals[ki * 4 + m])

        def _step(u, e, prefetch, drain_prev):
            b0, b1, ob = 2 * e, 2 * e + 1, e
            pltpu.make_async_copy(
                pairs_hbm.at[idx_v.at[2 * u]], gbuf[b0], gsem[b0]).wait()
            pltpu.make_async_copy(
                pairs_hbm.at[idx_v.at[2 * u + 1]], gbuf[b1], gsem[b1]).wait()

            if drain_prev:
                pltpu.make_async_copy(
                    obuf[ob],
                    out_hbm.at[pl.ds(i0, _CHUNK), pl.ds((u - 2) * 128, 128)],
                    ssem[ob]).wait()

            _assemble(u, 0, gbuf[b0], obuf[ob])
            _assemble(u, 1, gbuf[b1], obuf[ob])

            pltpu.async_copy(
                obuf[ob],
                out_hbm.at[pl.ds(i0, _CHUNK), pl.ds(u * 128, 128)],
                ssem[ob])

            if prefetch:
                def _fire(u=u, b0=b0, b1=b1):
                    pltpu.async_copy(
                        pairs_hbm.at[idx_v.at[2 * u + 4]], gbuf[b0], gsem[b0])
                    pltpu.async_copy(
                        pairs_hbm.at[idx_v.at[2 * u + 5]], gbuf[b1], gsem[b1])

                if isinstance(u, int):
                    _fire()
                else:
                    pl.when(u < npair - 2)(_fire)

        _step(0, 0, True, False)
        _step(1, 1, True, False)

        @pl.loop(1, npair // 2)
        def _round(t):
            for e in range(2):
                _step(2 * t + e, e, True, True)

        _step(npair - 1, (npair - 1) % 2, False, True)

        for u in (npair - 2, npair - 1):
            pltpu.make_async_copy(
                obuf[u % 2],
                out_hbm.at[pl.ds(i0, _CHUNK), pl.ds(u * 128, 128)],
                ssem[u % 2]).wait()

    return emb_kernel


def _make_relayout(vocab):
    nblk = vocab // 128
    tail = vocab - nblk * 128
    per_w = nblk // _NW
    extra = nblk - per_w * _NW
    mesh = plsc.VectorSubcoreMesh(
        core_axis_name="c", subcore_axis_name="s",
        num_cores=_NC, num_subcores=_NS,
    )

    @functools.partial(
        pl.kernel,
        out_type=jax.ShapeDtypeStruct((vocab // 2, 2 * _HID), jnp.float32),
        mesh=mesh,
        scratch_types=(
            [pltpu.VMEM((_HID, 128), jnp.float32) for _ in range(2)]
            + [pltpu.VMEM((_HID, 128), jnp.float32) for _ in range(2)]
            + [pltpu.SemaphoreType.DMA for _ in range(4)]
        ),
        compiler_params=pltpu.CompilerParams(needs_layout_passes=False),
    )
    def relayout_kernel(tt_hbm, tailp_hbm, pairs_hbm, ib0, ib1, ob0, ob1,
                        *sems):
        ibuf = (ib0, ib1)
        obuf = (ob0, ob1)
        gsem = sems[0:2]
        ssem = sems[2:4]
        wid = lax.axis_index("s") * _NC + lax.axis_index("c")
        start = wid * per_w + jnp.minimum(wid, extra)
        rows = [lax.iota(jnp.int32, _L) + 16 * mm for mm in range(4)]

        def _load(b, e):
            pltpu.async_copy(
                tt_hbm.at[:, pl.ds(b * 128, 128)], ibuf[e], gsem[e])

        def _wait_load(b, e):
            pltpu.make_async_copy(
                tt_hbm.at[:, pl.ds(b * 128, 128)], ibuf[e], gsem[e]).wait()

        def _store(b, e):
            pltpu.async_copy(
                obuf[e], pairs_hbm.at[pl.ds(b * 64, 64), :], ssem[e])

        def _wait_store(b, e):
            pltpu.make_async_copy(
                obuf[e], pairs_hbm.at[pl.ds(b * 64, 64), :], ssem[e]).wait()

        def _transpose(src, dst, nq):
            @plsc.parallel_loop(0, nq // 4, unroll=2)
            def _qgrp(qg):
                for q2 in range(2):
                    vals = []
                    for qq in range(2):
                        q = qg * 4 + q2 * 2 + qq
                        c0 = jnp.full((_L,), 2 * q, jnp.int32)
                        for m in range(8):
                            vals.append(plsc.load_gather(
                                src, [rows[m % 4], c0 + (m // 4)]))
                    for qq in range(2):
                        q = qg * 4 + q2 * 2 + qq
                        for m in range(8):
                            dst[q, pl.ds(16 * m, _L)] = vals[qq * 8 + m]

        nround = per_w // 2
        _load(start, 0)
        _load(start + 1, 1)

        @pl.loop(0, nround)
        def _round(t):
            for e in range(2):
                b = start + 2 * t + e
                _wait_load(b, e)

                @pl.when(t >= 1)
                def _(b=b, e=e):
                    _wait_store(b - 2, e)

                _transpose(ibuf[e], obuf[e], 64)
                _store(b, e)

                @pl.when(t < nround - 1)
                def _(b=b, e=e):
                    _load(b + 2, e)

        for e in range(2):
            _wait_store(start + per_w - 2 + e, e)

        @pl.when(wid < extra)
        def _():
            b = start + per_w
            pltpu.sync_copy(tt_hbm.at[:, pl.ds(b * 128, 128)], ibuf[0])
            _transpose(ibuf[0], obuf[0], 64)
            pltpu.sync_copy(obuf[0], pairs_hbm.at[pl.ds(b * 64, 64), :])

        if tail:
            @pl.when(wid == _NW - 1)
            def _():
                pltpu.sync_copy(tailp_hbm, obuf[1].at[pl.ds(0, tail // 2), :])
                pltpu.sync_copy(
                    obuf[1].at[pl.ds(0, tail // 2), :],
                    pairs_hbm.at[pl.ds(nblk * 64, tail // 2), :])

    return relayout_kernel


def kernel(x, emb_table):
    nrows, ncols = x.shape
    vocab, hid = emb_table.shape
    assert nrows == _NW * _CHUNK and hid == _HID
    xt = jnp.swapaxes(x.astype(jnp.int32), 0, 1)
    tt = jnp.swapaxes(emb_table, 0, 1)
    nblk = vocab // 128
    tailp = emb_table[nblk * 128:].reshape(-1, 2 * _HID)
    pairs = _make_relayout(vocab)(tt, tailp)
    out = _make_kernel(nrows, ncols)(pairs, xt)
    return out.reshape(nrows, ncols, _HID)


# speedup vs baseline: 2.3593x; 1.6099x over previous
import functools

import jax
import jax.numpy as jnp
from jax import lax
from jax.experimental import pallas as pl
from jax.experimental.pallas import tpu as pltpu
from jax.experimental.pallas import tpu_sc as plsc

_HID = 64
_SCALE = 8.0

_NC = 2
_NS = 16
_NW = _NC * _NS
_L = 16

_CHUNK = 128


def _make_kernel(nrows, ncols):
    npair = ncols // 2
    assert npair * 2 == ncols
    mesh = plsc.VectorSubcoreMesh(
        core_axis_name="c", subcore_axis_name="s",
        num_cores=_NC, num_subcores=_NS,
    )

    @functools.partial(
        pl.kernel,
        out_type=jax.ShapeDtypeStruct((nrows, ncols * _HID), jnp.float32),
        mesh=mesh,
        scratch_types=(
            [
                pltpu.VMEM((ncols, _CHUNK), jnp.int32),
                pltpu.VMEM((ncols, _CHUNK), jnp.int32),
            ]
            + [pltpu.VMEM((_CHUNK, 128), jnp.float32) for _ in range(4)]
            + [pltpu.VMEM((_CHUNK, 128), jnp.float32) for _ in range(2)]
            + [pltpu.SemaphoreType.DMA for _ in range(4)]
            + [pltpu.SemaphoreType.DMA for _ in range(2)]
        ),
        compiler_params=pltpu.CompilerParams(needs_layout_passes=False),
    )
    def emb_kernel(pairs_hbm, idxt_hbm, out_hbm, idx_v, par_v, *scratch):
        gbuf = scratch[0:4]
        obuf = scratch[4:6]
        gsem = scratch[6:10]
        ssem = scratch[10:12]
        wid = lax.axis_index("s") * _NC + lax.axis_index("c")
        i0 = wid * _CHUNK
        pltpu.sync_copy(idxt_hbm.at[:, pl.ds(i0, _CHUNK)], idx_v)

        @pl.loop(0, ncols)
        def _split(j):
            for k in range(_CHUNK // _L):
                sl = pl.ds(k * _L, _L)
                v = idx_v[j, sl]
                idx_v[j, sl] = v >> 1
                par_v[j, sl] = (v & 1) * _HID

        for j in range(4):
            pltpu.async_copy(pairs_hbm.at[idx_v.at[j]], gbuf[j], gsem[j])

        def _assemble(u, h, src, dst):
            j = 2 * u + h

            @pl.loop(0, _CHUNK // _L)
            def _rowgrp(g):
                r0 = g * _L
                parv = par_v[j, pl.ds(r0, _L)]
                for k2 in range(_L // 4):
                    vals = []
                    for k in range(k2 * 4, k2 * 4 + 4):
                        off = parv[k]
                        for m in range(_HID // _L):
                            vals.append(
                                src[r0 + k, pl.ds(off + m * _L, _L)]
                                * _SCALE)
                    for ki in range(4):
                        k = k2 * 4 + ki
                        for m in range(_HID // _L):
                            dst[r0 + k, pl.ds(h * _HID + m * _L, _L)] = (
                                vals[ki * 4 + m])

        def _step(u, e, prefetch, drain_prev):
            b0, b1, ob = 2 * e, 2 * e + 1, e
            pltpu.make_async_copy(
                pairs_hbm.at[idx_v.at[2 * u]], gbuf[b0], gsem[b0]).wait()
            pltpu.make_async_copy(
                pairs_hbm.at[idx_v.at[2 * u + 1]], gbuf[b1], gsem[b1]).wait()

            if drain_prev:
                pltpu.make_async_copy(
                    obuf[ob],
                    out_hbm.at[pl.ds(i0, _CHUNK), pl.ds((u - 2) * 128, 128)],
                    ssem[ob]).wait()

            _assemble(u, 0, gbuf[b0], obuf[ob])
            _assemble(u, 1, gbuf[b1], obuf[ob])

            pltpu.async_copy(
                obuf[ob],
                out_hbm.at[pl.ds(i0, _CHUNK), pl.ds(u * 128, 128)],
                ssem[ob])

            if prefetch:
                def _fire(u=u, b0=b0, b1=b1):
                    pltpu.async_copy(
                        pairs_hbm.at[idx_v.at[2 * u + 4]], gbuf[b0], gsem[b0])
                    pltpu.async_copy(
                        pairs_hbm.at[idx_v.at[2 * u + 5]], gbuf[b1], gsem[b1])

                if isinstance(u, int):
                    _fire()
                else:
                    pl.when(u < npair - 2)(_fire)

        _step(0, 0, True, False)
        _step(1, 1, True, False)

        @pl.loop(1, npair // 2)
        def _round(t):
            for e in range(2):
                _step(2 * t + e, e, True, True)

        _step(npair - 1, (npair - 1) % 2, False, True)

        for u in (npair - 2, npair - 1):
            pltpu.make_async_copy(
                obuf[u % 2],
                out_hbm.at[pl.ds(i0, _CHUNK), pl.ds(u * 128, 128)],
                ssem[u % 2]).wait()

    return emb_kernel


def _make_relayout(vocab):
    nblk = vocab // 128
    tail = vocab - nblk * 128
    per_w = nblk // _NW
    extra = nblk - per_w * _NW
    mesh = plsc.VectorSubcoreMesh(
        core_axis_name="c", subcore_axis_name="s",
        num_cores=_NC, num_subcores=_NS,
    )

    @functools.partial(
        pl.kernel,
        out_type=jax.ShapeDtypeStruct((vocab // 2, 2 * _HID), jnp.float32),
        mesh=mesh,
        scratch_types=(
            [pltpu.VMEM((_HID, 128), jnp.float32) for _ in range(2)]
            + [pltpu.VMEM((_HID, 128), jnp.float32) for _ in range(2)]
            + [pltpu.SemaphoreType.DMA for _ in range(4)]
        ),
        compiler_params=pltpu.CompilerParams(needs_layout_passes=False),
    )
    def relayout_kernel(tt_hbm, tailp_hbm, pairs_hbm, ib0, ib1, ob0, ob1,
                        *sems):
        ibuf = (ib0, ib1)
        obuf = (ob0, ob1)
        gsem = sems[0:2]
        ssem = sems[2:4]
        wid = lax.axis_index("s") * _NC + lax.axis_index("c")
        start = wid * per_w + jnp.minimum(wid, extra)
        rows = [lax.iota(jnp.int32, _L) + 16 * mm for mm in range(4)]

        def _load(b, e):
            pltpu.async_copy(
                tt_hbm.at[:, pl.ds(b * 128, 128)], ibuf[e], gsem[e])

        def _wait_load(b, e):
            pltpu.make_async_copy(
                tt_hbm.at[:, pl.ds(b * 128, 128)], ibuf[e], gsem[e]).wait()

        def _store(b, e):
            pltpu.async_copy(
                obuf[e], pairs_hbm.at[pl.ds(b * 64, 64), :], ssem[e])

        def _wait_store(b, e):
            pltpu.make_async_copy(
                obuf[e], pairs_hbm.at[pl.ds(b * 64, 64), :], ssem[e]).wait()

        def _transpose(src, dst, nq):
            @plsc.parallel_loop(0, nq // 4, unroll=2)
            def _qgrp(qg):
                for q2 in range(2):
                    vals = []
                    for qq in range(2):
                        q = qg * 4 + q2 * 2 + qq
                        c0 = jnp.full((_L,), 2 * q, jnp.int32)
                        for m in range(8):
                            vals.append(plsc.load_gather(
                                src, [rows[m % 4], c0 + (m // 4)]))
                    for qq in range(2):
                        q = qg * 4 + q2 * 2 + qq
                        for m in range(8):
                            dst[q, pl.ds(16 * m, _L)] = vals[qq * 8 + m]

        nround = per_w // 2
        _load(start, 0)
        _load(start + 1, 1)

        @pl.loop(0, nround)
        def _round(t):
            for e in range(2):
                b = start + 2 * t + e
                _wait_load(b, e)

                @pl.when(t >= 1)
                def _(b=b, e=e):
                    _wait_store(b - 2, e)

                _transpose(ibuf[e], obuf[e], 64)
                _store(b, e)

                @pl.when(t < nround - 1)
                def _(b=b, e=e):
                    _load(b + 2, e)

        for e in range(2):
            _wait_store(start + per_w - 2 + e, e)

        @pl.when(wid < extra)
        def _():
            b = start + per_w
            pltpu.sync_copy(tt_hbm.at[:, pl.ds(b * 128, 128)], ibuf[0])
            _transpose(ibuf[0], obuf[0], 64)
            pltpu.sync_copy(obuf[0], pairs_hbm.at[pl.ds(b * 64, 64), :])

        if tail:
            @pl.when(wid == _NW - 1)
            def _():
                pltpu.sync_copy(tailp_hbm, obuf[1].at[pl.ds(0, tail // 2), :])
                pltpu.sync_copy(
                    obuf[1].at[pl.ds(0, tail // 2), :],
                    pairs_hbm.at[pl.ds(nblk * 64, tail // 2), :])

    return relayout_kernel


def kernel(x, emb_table):
    nrows, ncols = x.shape
    vocab, hid = emb_table.shape
    assert nrows == _NW * _CHUNK and hid == _HID
    xt = jnp.swapaxes(x.astype(jnp.int32), 0, 1)
    pairs = emb_table.reshape(vocab // 2, 2 * _HID)
    out = _make_kernel(nrows, ncols)(pairs, xt)
    return out.reshape(nrows, ncols, _HID)
